# baseline (speedup 1.0000x reference)
import functools

import jax
import jax.numpy as jnp
from jax.experimental import pallas as pl
from jax.experimental.pallas import tpu as pltpu

S = 2048
D_MODEL = 1024
D_SPACE = 128
N_NEURONS = 4096
N_CLUSTERS = 64
CLUSTER_SIZE = N_NEURONS // N_CLUSTERS
K_CLUSTER = 2
MAX_K = 64
T_BLK = 256
GRID = S // T_BLK


def _fused_body(x_ref, proj_ref, pb_ref, tauk_ref, taub_ref, cemb_ref,
                nemb_ref, kn_ref, out_ref, cfreq_ref, nfreq_ref,
                caux_ref, naux_ref):
    i = pl.program_id(0)

    xb = x_ref[...]
    hb = jnp.dot(xb, proj_ref[...],
                 preferred_element_type=jnp.float32) + pb_ref[...]
    taub = jnp.sum(xb * tauk_ref[...], axis=-1, keepdims=True) \
        + taub_ref[...]

    ce = cemb_ref[...]
    ce_n = ce / (jnp.sqrt(jnp.sum(ce * ce, axis=-1, keepdims=True)) + 1e-08)
    cs = jax.lax.dot_general(hb, ce_n, (((1,), (1,)), ((), ())),
                             preferred_element_type=jnp.float32)
    m = jnp.max(cs, axis=-1, keepdims=True)
    p = jnp.exp(cs - m)
    p = p / jnp.sum(p, axis=-1, keepdims=True)

    @pl.when(i == 0)
    def _():
        cfreq_ref[...] = jnp.zeros_like(cfreq_ref)
        nfreq_ref[...] = jnp.zeros_like(nfreq_ref)
    cfreq_ref[...] += jnp.sum(p, axis=0, keepdims=True)

    lane64 = jax.lax.broadcasted_iota(jnp.int32, (T_BLK, N_CLUSTERS), 1)
    big = jnp.int32(N_CLUSTERS + 1)
    a1 = jnp.min(jnp.where(cs == m, lane64, big), axis=-1, keepdims=True)
    cs2 = jnp.where(lane64 == a1, -jnp.inf, cs)
    m2 = jnp.max(cs2, axis=-1, keepdims=True)
    a2 = jnp.min(jnp.where(cs2 == m2, lane64, big), axis=-1, keepdims=True)

    ne = nemb_ref[...]
    inv_n = 1.0 / (jnp.sqrt(jnp.sum(ne * ne, axis=-1)) + 1e-08)
    s_all = jax.lax.dot_general(hb, ne, (((1,), (1,)), ((), ())),
                                preferred_element_type=jnp.float32)
    s_all = s_all * inv_n[None, :]

    acc1 = jnp.zeros((T_BLK, CLUSTER_SIZE), jnp.float32)
    acc2 = jnp.zeros((T_BLK, CLUSTER_SIZE), jnp.float32)
    for c in range(N_CLUSTERS):
        seg = s_all[:, c * CLUSTER_SIZE:(c + 1) * CLUSTER_SIZE]
        acc1 = acc1 + jnp.where(a1 == c, seg, 0.0)
        acc2 = acc2 + jnp.where(a2 == c, seg, 0.0)
    a_sc = jnp.concatenate([acc1, acc2], axis=1)

    raw_g = a_sc - taub
    gate_g = jnp.where(raw_g > 0, raw_g, 1e-08 * jnp.exp(raw_g))
    e_g = jnp.exp(gate_g) - 1.0

    bits = jax.lax.bitcast_convert_type(e_g, jnp.int32)
    thr_bits = jnp.zeros((T_BLK, 1), jnp.int32)
    for b in range(30, -1, -1):
        cand = thr_bits | jnp.int32(1 << b)
        cnt = jnp.sum((bits >= cand).astype(jnp.int32), axis=-1, keepdims=True)
        thr_bits = jnp.where(cnt >= MAX_K, cand, thr_bits)
    thr = jax.lax.bitcast_convert_type(thr_bits, jnp.float32)

    keep_g = e_g >= thr
    e_kept = jnp.where(keep_g, e_g, 0.0)
    gsum = jnp.sum(e_kept, axis=-1, keepdims=True) + 1e-08
    gstr = jnp.tanh(jnp.max(e_kept, axis=-1, keepdims=True))

    lane_all = jax.lax.broadcasted_iota(jnp.int32, (T_BLK, N_NEURONS), 1)
    cid = jax.lax.shift_right_logical(lane_all, 6)
    active = (cid == a1) | (cid == a2)
    raw_d = s_all - taub
    gate_d = jnp.where(raw_d > 0, raw_d, 1e-08 * jnp.exp(raw_d))
    e_d = jnp.exp(gate_d) - 1.0
    gates = jnp.where(active & (e_d >= thr), e_d / gsum * gstr, 0.0)

    nfreq_ref[...] += jnp.sum(gates, axis=0, keepdims=True)

    kn = kn_ref[...]
    act = jax.lax.dot_general(xb.astype(jnp.bfloat16), kn,
                              (((1,), (1,)), ((), ())),
                              preferred_element_type=jnp.float32)
    gated = (act * gates).astype(jnp.bfloat16)
    out_ref[...] = jnp.dot(gated, kn, preferred_element_type=jnp.float32)

    @pl.when(i == GRID - 1)
    def _():
        cfreq = cfreq_ref[...] * (1.0 / S)
        caux_ref[...] = jnp.sum((cfreq - 1.0 / N_CLUSTERS) ** 2,
                                keepdims=True) * N_CLUSTERS
        nfreq = nfreq_ref[...] * (1.0 / S)
        naux_ref[...] = jnp.sum((nfreq - 1.0 / N_NEURONS) ** 2,
                                keepdims=True) * N_NEURONS


@jax.jit
def kernel(x, proj_kernel, proj_bias, tau_kernel, tau_bias,
           neuron_emb, cluster_emb, know_neurons):
    x2d = x.reshape(S, D_MODEL)
    grid_spec = pl.GridSpec(
        grid=(GRID,),
        in_specs=[
            pl.BlockSpec((T_BLK, D_MODEL), lambda i: (i, 0)),
            pl.BlockSpec((D_MODEL, D_SPACE), lambda i: (0, 0)),
            pl.BlockSpec((1, D_SPACE), lambda i: (0, 0)),
            pl.BlockSpec((1, D_MODEL), lambda i: (0, 0)),
            pl.BlockSpec((1, 1), lambda i: (0, 0)),
            pl.BlockSpec((N_CLUSTERS, D_SPACE), lambda i: (0, 0)),
            pl.BlockSpec((N_NEURONS, D_SPACE), lambda i: (0, 0)),
            pl.BlockSpec((N_NEURONS, D_MODEL), lambda i: (0, 0)),
        ],
        out_specs=[
            pl.BlockSpec((T_BLK, D_MODEL), lambda i: (i, 0)),
            pl.BlockSpec((1, N_CLUSTERS), lambda i: (0, 0)),
            pl.BlockSpec((1, N_NEURONS), lambda i: (0, 0)),
            pl.BlockSpec((1, 1), lambda i: (0, 0)),
            pl.BlockSpec((1, 1), lambda i: (0, 0)),
        ],
    )
    out, _, _, caux, naux = pl.pallas_call(
        _fused_body,
        grid_spec=grid_spec,
        out_shape=[
            jax.ShapeDtypeStruct((S, D_MODEL), jnp.float32),
            jax.ShapeDtypeStruct((1, N_CLUSTERS), jnp.float32),
            jax.ShapeDtypeStruct((1, N_NEURONS), jnp.float32),
            jax.ShapeDtypeStruct((1, 1), jnp.float32),
            jax.ShapeDtypeStruct((1, 1), jnp.float32),
        ],
        compiler_params=pltpu.CompilerParams(
            dimension_semantics=("arbitrary",),
        ),
    )(x2d, proj_kernel, proj_bias.reshape(1, D_SPACE),
      tau_kernel.reshape(1, D_MODEL), tau_bias.reshape(1, 1),
      cluster_emb, neuron_emb, know_neurons.astype(jnp.bfloat16))
    return (out.reshape(1, S, D_MODEL), caux.reshape(()), naux.reshape(()))


# speedup vs baseline: 1.0998x; 1.0998x over previous
import jax
import jax.numpy as jnp
from jax.experimental import pallas as pl
from jax.experimental.pallas import tpu as pltpu

S = 2048
D_MODEL = 1024
D_SPACE = 128
N_NEURONS = 4096
N_CLUSTERS = 64
CLUSTER_SIZE = N_NEURONS // N_CLUSTERS
MAX_K = 64
T_BLK = 256
GRID = S // T_BLK


def _fused_body(x_ref, proj_ref, pb_ref, tauk_ref, taub_ref, cemb_ref,
                nemb_ref, kn_ref, fold_ref, out_ref, cfreq_ref, nfreq_ref,
                caux_ref, naux_ref, nen_ref, cid_ref):
    i = pl.program_id(0)

    @pl.when(i == 0)
    def _():
        ne = nemb_ref[...]
        inv_n = 1.0 / (jnp.sqrt(jnp.sum(ne * ne, axis=-1, keepdims=True))
                       + 1e-08)
        nen_ref[...] = ne * inv_n
        lane = jax.lax.broadcasted_iota(jnp.int32, (1, N_NEURONS), 1)
        cid_ref[...] = jax.lax.shift_right_logical(lane, 6)
        cfreq_ref[...] = jnp.zeros_like(cfreq_ref)
        nfreq_ref[...] = jnp.zeros_like(nfreq_ref)

    xb = x_ref[...]
    hb = jnp.dot(xb, proj_ref[...],
                 preferred_element_type=jnp.float32) + pb_ref[...]
    taub = jnp.sum(xb * tauk_ref[...], axis=-1, keepdims=True) \
        + taub_ref[...]

    ce = cemb_ref[...]
    ce_n = ce / (jnp.sqrt(jnp.sum(ce * ce, axis=-1, keepdims=True)) + 1e-08)
    cs = jax.lax.dot_general(hb, ce_n, (((1,), (1,)), ((), ())),
                             preferred_element_type=jnp.float32)
    m = jnp.max(cs, axis=-1, keepdims=True)
    p = jnp.exp(cs - m)
    p = p / jnp.sum(p, axis=-1, keepdims=True)
    cfreq_ref[...] += jnp.sum(p, axis=0, keepdims=True)

    lane64 = jax.lax.broadcasted_iota(jnp.int32, (T_BLK, N_CLUSTERS), 1)
    big = jnp.int32(N_CLUSTERS + 1)
    a1 = jnp.min(jnp.where(cs == m, lane64, big), axis=-1, keepdims=True)
    cs2 = jnp.where(lane64 == a1, -jnp.inf, cs)
    m2 = jnp.max(cs2, axis=-1, keepdims=True)
    a2 = jnp.min(jnp.where(cs2 == m2, lane64, big), axis=-1, keepdims=True)

    s_all = jax.lax.dot_general(hb, nen_ref[...], (((1,), (1,)), ((), ())),
                                preferred_element_type=jnp.float32)

    cid = cid_ref[...]
    m1 = cid == a1
    m2m = cid == a2
    fold = fold_ref[...]
    b1 = jnp.where(m1, s_all, 0.0)
    b2 = jnp.where(m2m, s_all, 0.0)
    g1s = jax.lax.dot_general(b1, fold, (((1,), (0,)), ((), ())),
                              preferred_element_type=jnp.float32)
    g2s = jax.lax.dot_general(b2, fold, (((1,), (0,)), ((), ())),
                              preferred_element_type=jnp.float32)
    a_sc = jnp.concatenate([g1s, g2s], axis=1)

    raw_g = a_sc - taub
    gate_g = jnp.where(raw_g > 0, raw_g, 1e-08 * jnp.exp(raw_g))
    e_g = jnp.exp(gate_g) - 1.0

    bits = jax.lax.bitcast_convert_type(e_g, jnp.int32)
    thr_bits = jnp.zeros((T_BLK, 1), jnp.int32)
    for b in range(30, -1, -1):
        cand = thr_bits | jnp.int32(1 << b)
        cnt = jnp.sum((bits >= cand).astype(jnp.int32), axis=-1, keepdims=True)
        thr_bits = jnp.where(cnt >= MAX_K, cand, thr_bits)
    thr = jax.lax.bitcast_convert_type(thr_bits, jnp.float32)

    e_kept = jnp.where(e_g >= thr, e_g, 0.0)
    gsum = jnp.sum(e_kept, axis=-1, keepdims=True) + 1e-08
    gstr = jnp.tanh(jnp.max(e_kept, axis=-1, keepdims=True))
    g_val = e_kept * (gstr / gsum)

    g1 = g_val[:, :CLUSTER_SIZE]
    g2 = g_val[:, CLUSTER_SIZE:]

    p1 = jnp.where(lane64 == a1, 1.0, 0.0)
    p2 = jnp.where(lane64 == a2, 1.0, 0.0)
    nfreq_ref[...] += (
        jax.lax.dot_general(p1, g1, (((0,), (0,)), ((), ())),
                            preferred_element_type=jnp.float32)
        + jax.lax.dot_general(p2, g2, (((0,), (0,)), ((), ())),
                              preferred_element_type=jnp.float32))

    g1x = jax.lax.dot_general(g1, fold, (((1,), (1,)), ((), ())),
                              preferred_element_type=jnp.float32)
    g2x = jax.lax.dot_general(g2, fold, (((1,), (1,)), ((), ())),
                              preferred_element_type=jnp.float32)
    kn = kn_ref[...]
    act = jax.lax.dot_general(xb, kn, (((1,), (1,)), ((), ())),
                              preferred_element_type=jnp.float32)
    gated = act * (jnp.where(m1, g1x, 0.0) + jnp.where(m2m, g2x, 0.0))
    out_ref[...] = jnp.dot(gated, kn, preferred_element_type=jnp.float32)

    @pl.when(i == GRID - 1)
    def _():
        cfreq = cfreq_ref[...] * (1.0 / S)
        caux_ref[...] = jnp.sum((cfreq - 1.0 / N_CLUSTERS) ** 2,
                                keepdims=True) * N_CLUSTERS
        nfreq = nfreq_ref[...] * (1.0 / S)
        naux_ref[...] = jnp.sum((nfreq - 1.0 / N_NEURONS) ** 2,
                                keepdims=True)[:1, :1] * N_NEURONS


@jax.jit
def kernel(x, proj_kernel, proj_bias, tau_kernel, tau_bias,
           neuron_emb, cluster_emb, know_neurons):
    x2d = x.reshape(S, D_MODEL)
    fold = (jnp.arange(N_NEURONS, dtype=jnp.int32)[:, None] % CLUSTER_SIZE
            == jnp.arange(CLUSTER_SIZE, dtype=jnp.int32)[None, :]
            ).astype(jnp.float32)
    in_specs = [
            pl.BlockSpec((T_BLK, D_MODEL), lambda i: (i, 0)),
            pl.BlockSpec((D_MODEL, D_SPACE), lambda i: (0, 0)),
            pl.BlockSpec((1, D_SPACE), lambda i: (0, 0)),
            pl.BlockSpec((1, D_MODEL), lambda i: (0, 0)),
            pl.BlockSpec((1, 1), lambda i: (0, 0)),
            pl.BlockSpec((N_CLUSTERS, D_SPACE), lambda i: (0, 0)),
            pl.BlockSpec((N_NEURONS, D_SPACE), lambda i: (0, 0)),
            pl.BlockSpec((N_NEURONS, D_MODEL), lambda i: (0, 0)),
            pl.BlockSpec((N_NEURONS, CLUSTER_SIZE), lambda i: (0, 0)),
    ]
    out_specs = [
            pl.BlockSpec((T_BLK, D_MODEL), lambda i: (i, 0)),
            pl.BlockSpec((1, N_CLUSTERS), lambda i: (0, 0)),
            pl.BlockSpec((N_CLUSTERS, CLUSTER_SIZE), lambda i: (0, 0)),
            pl.BlockSpec((1, 1), lambda i: (0, 0)),
            pl.BlockSpec((1, 1), lambda i: (0, 0)),
    ]
    out, _, _, caux, naux = pl.pallas_call(
        _fused_body,
        grid=(GRID,),
        in_specs=in_specs,
        out_specs=out_specs,
        out_shape=[
            jax.ShapeDtypeStruct((S, D_MODEL), jnp.float32),
            jax.ShapeDtypeStruct((1, N_CLUSTERS), jnp.float32),
            jax.ShapeDtypeStruct((N_CLUSTERS, CLUSTER_SIZE), jnp.float32),
            jax.ShapeDtypeStruct((1, 1), jnp.float32),
            jax.ShapeDtypeStruct((1, 1), jnp.float32),
        ],
        scratch_shapes=[
            pltpu.VMEM((N_NEURONS, D_SPACE), jnp.float32),
            pltpu.VMEM((1, N_NEURONS), jnp.int32),
        ],
        compiler_params=pltpu.CompilerParams(
            dimension_semantics=("arbitrary",),
        ),
    )(x2d, proj_kernel, proj_bias.reshape(1, D_SPACE),
      tau_kernel.reshape(1, D_MODEL), tau_bias.reshape(1, 1),
      cluster_emb, neuron_emb, know_neurons, fold)
    return (out.reshape(1, S, D_MODEL), caux.reshape(()), naux.reshape(()))
